# baseline (speedup 1.0000x reference)
import functools

import jax
import jax.numpy as jnp
from jax import lax
from jax.experimental import pallas as pl
from jax.experimental.pallas import tpu as pltpu
from jax.experimental.pallas import tpu_sc as plsc

_NC = 2
_NS = 16
_NW = _NC * _NS


def _sc_concat_embed(table, idx, feat2, dst_idx, T):
    B, = idx.shape
    V, D = table.shape
    b_per_w = B // _NW
    n_rows = B * (T + 1)
    mesh = plsc.VectorSubcoreMesh(
        core_axis_name="c", subcore_axis_name="s",
        num_cores=_NC, num_subcores=_NS,
    )

    @functools.partial(
        pl.kernel,
        out_type=jax.ShapeDtypeStruct((n_rows, D), table.dtype),
        mesh=mesh,
        compiler_params=pltpu.CompilerParams(use_tc_tiling_on_sc=False),
        scratch_types=[
            pltpu.VMEM((b_per_w,), jnp.int32),
            pltpu.VMEM((b_per_w,), jnp.int32),
            pltpu.VMEM((b_per_w, D), jnp.float32),
            pltpu.SemaphoreType.DMA,
            pltpu.SemaphoreType.DMA,
        ],
    )
    def body(table_hbm, idx_hbm, feat_hbm, dst_hbm, out_hbm,
             idx_v, dst_v, rows_v, gsem, csem):
        wid = lax.axis_index("s") * _NC + lax.axis_index("c")
        base = wid * b_per_w
        pltpu.sync_copy(idx_hbm.at[pl.ds(base, b_per_w)], idx_v)
        pltpu.sync_copy(dst_hbm.at[pl.ds(base, b_per_w)], dst_v)
        gather = pltpu.async_copy(table_hbm.at[idx_v], rows_v, gsem)
        copies = []
        for b in range(b_per_w):
            bb = base + b
            copies.append(pltpu.async_copy(
                feat_hbm.at[pl.ds(bb * T, T)],
                out_hbm.at[pl.ds(bb * (T + 1) + 1, T)],
                csem,
            ))
        gather.wait()
        pltpu.async_copy(rows_v, out_hbm.at[dst_v], gsem).wait()
        for c in copies:
            c.wait()

    return body(table, idx, feat2, dst_idx)


def kernel(feature, index_value, embedding_table):
    B, T, D = feature.shape
    idx = index_value.astype(jnp.int32)
    dst_idx = (jnp.arange(B, dtype=jnp.int32) * (T + 1)).astype(jnp.int32)
    feat2 = feature.reshape(B * T, D)
    out2 = _sc_concat_embed(embedding_table, idx, feat2, dst_idx, T)
    return out2.reshape(B, T + 1, D)


# speedup vs baseline: 21.3207x; 21.3207x over previous
import functools

import jax
import jax.numpy as jnp
from jax import lax
from jax.experimental import pallas as pl
from jax.experimental.pallas import tpu as pltpu
from jax.experimental.pallas import tpu_sc as plsc

_NC = 2
_NS = 16
_NW = _NC * _NS


def _sc_gather(table, idx):
    B, = idx.shape
    V, D = table.shape
    b_per_w = B // _NW
    mesh = plsc.VectorSubcoreMesh(
        core_axis_name="c", subcore_axis_name="s",
        num_cores=_NC, num_subcores=_NS,
    )

    @functools.partial(
        pl.kernel,
        out_type=jax.ShapeDtypeStruct((B, D), table.dtype),
        mesh=mesh,
        scratch_types=[
            pltpu.VMEM((b_per_w,), jnp.int32),
            pltpu.VMEM((b_per_w, D), jnp.float32),
            pltpu.SemaphoreType.DMA,
        ],
    )
    def gather_kernel(table_hbm, idx_hbm, out_hbm, idx_v, rows_v, sem):
        wid = lax.axis_index("s") * _NC + lax.axis_index("c")
        base = wid * b_per_w
        pltpu.sync_copy(idx_hbm.at[pl.ds(base, b_per_w)], idx_v)
        pltpu.async_copy(table_hbm.at[idx_v], rows_v, sem).wait()
        pltpu.sync_copy(rows_v, out_hbm.at[pl.ds(base, b_per_w)])

    return gather_kernel(table, idx)


def _concat_body(emb_ref, feat_ref, out_ref):
    out_ref[:, 0:1, :] = emb_ref[...]
    out_ref[:, 1:, :] = feat_ref[...]


def _tc_concat(emb, feature, block_b=32):
    B, T, D = feature.shape
    emb3 = emb.reshape(B, 1, D)
    return pl.pallas_call(
        _concat_body,
        grid=(B // block_b,),
        in_specs=[
            pl.BlockSpec((block_b, 1, D), lambda b: (b, 0, 0)),
            pl.BlockSpec((block_b, T, D), lambda b: (b, 0, 0)),
        ],
        out_specs=pl.BlockSpec((block_b, T + 1, D), lambda b: (b, 0, 0)),
        out_shape=jax.ShapeDtypeStruct((B, T + 1, D), feature.dtype),
    )(emb3, feature)


def kernel(feature, index_value, embedding_table):
    idx = index_value.astype(jnp.int32)
    emb = _sc_gather(embedding_table, idx)
    return _tc_concat(emb, feature)


# speedup vs baseline: 21.6567x; 1.0158x over previous
import functools

import jax
import jax.numpy as jnp
from jax import lax
from jax.experimental import pallas as pl
from jax.experimental.pallas import tpu as pltpu
from jax.experimental.pallas import tpu_sc as plsc

_NC = 2
_NS = 16
_NW = _NC * _NS


def _sc_gather(table, idx):
    B, = idx.shape
    V, D = table.shape
    b_per_w = B // _NW
    mesh = plsc.VectorSubcoreMesh(
        core_axis_name="c", subcore_axis_name="s",
        num_cores=_NC, num_subcores=_NS,
    )

    @functools.partial(
        pl.kernel,
        out_type=jax.ShapeDtypeStruct((B, D), table.dtype),
        mesh=mesh,
        scratch_types=[
            pltpu.VMEM((b_per_w,), jnp.int32),
            pltpu.VMEM((b_per_w, D), jnp.float32),
            pltpu.SemaphoreType.DMA,
        ],
    )
    def gather_kernel(table_hbm, idx_hbm, out_hbm, idx_v, rows_v, sem):
        wid = lax.axis_index("s") * _NC + lax.axis_index("c")
        base = wid * b_per_w
        pltpu.sync_copy(idx_hbm.at[pl.ds(base, b_per_w)], idx_v)
        pltpu.async_copy(table_hbm.at[idx_v], rows_v, sem).wait()
        pltpu.sync_copy(rows_v, out_hbm.at[pl.ds(base, b_per_w)])

    return gather_kernel(table, idx)


def _concat_body(emb_ref, feat_ref, out_ref):
    out_ref[:, 0:1, :] = emb_ref[...]
    out_ref[:, 1:, :] = feat_ref[...]


def _tc_concat(emb, feature, block_b=64):
    B, T, D = feature.shape
    emb3 = emb.reshape(B, 1, D)
    return pl.pallas_call(
        _concat_body,
        grid=(B // block_b,),
        in_specs=[
            pl.BlockSpec((block_b, 1, D), lambda b: (b, 0, 0)),
            pl.BlockSpec((block_b, T, D), lambda b: (b, 0, 0)),
        ],
        out_specs=pl.BlockSpec((block_b, T + 1, D), lambda b: (b, 0, 0)),
        out_shape=jax.ShapeDtypeStruct((B, T + 1, D), feature.dtype),
    )(emb3, feature)


def kernel(feature, index_value, embedding_table):
    idx = index_value.astype(jnp.int32)
    emb = _sc_gather(embedding_table, idx)
    return _tc_concat(emb, feature)


# speedup vs baseline: 21.8605x; 1.0094x over previous
import functools

import jax
import jax.numpy as jnp
from jax import lax
from jax.experimental import pallas as pl
from jax.experimental.pallas import tpu as pltpu
from jax.experimental.pallas import tpu_sc as plsc

_NC = 2
_NS = 16
_NW = _NC * _NS


def _sc_gather(table, idx):
    B, = idx.shape
    V, D = table.shape
    b_per_w = B // _NW
    mesh = plsc.VectorSubcoreMesh(
        core_axis_name="c", subcore_axis_name="s",
        num_cores=_NC, num_subcores=_NS,
    )

    @functools.partial(
        pl.kernel,
        out_type=jax.ShapeDtypeStruct((B, D), table.dtype),
        mesh=mesh,
        scratch_types=[
            pltpu.VMEM((b_per_w,), jnp.int32),
            pltpu.VMEM((b_per_w, D), jnp.float32),
            pltpu.SemaphoreType.DMA,
        ],
    )
    def gather_kernel(table_hbm, idx_hbm, out_hbm, idx_v, rows_v, sem):
        wid = lax.axis_index("s") * _NC + lax.axis_index("c")
        base = wid * b_per_w
        pltpu.sync_copy(idx_hbm.at[pl.ds(base, b_per_w)], idx_v)
        pltpu.async_copy(table_hbm.at[idx_v], rows_v, sem).wait()
        pltpu.sync_copy(rows_v, out_hbm.at[pl.ds(base, b_per_w)])

    return gather_kernel(table, idx)


def _concat_body(emb_ref, feat_ref, out_ref):
    out_ref[:, 0:1, :] = emb_ref[...]
    out_ref[:, 1:, :] = feat_ref[...]


def _tc_concat(emb, feature, block_b=128):
    B, T, D = feature.shape
    emb3 = emb.reshape(B, 1, D)
    return pl.pallas_call(
        _concat_body,
        grid=(B // block_b,),
        in_specs=[
            pl.BlockSpec((block_b, 1, D), lambda b: (b, 0, 0)),
            pl.BlockSpec((block_b, T, D), lambda b: (b, 0, 0)),
        ],
        out_specs=pl.BlockSpec((block_b, T + 1, D), lambda b: (b, 0, 0)),
        out_shape=jax.ShapeDtypeStruct((B, T + 1, D), feature.dtype),
    )(emb3, feature)


def kernel(feature, index_value, embedding_table):
    idx = index_value.astype(jnp.int32)
    emb = _sc_gather(embedding_table, idx)
    return _tc_concat(emb, feature)
